# Initial kernel scaffold; baseline (speedup 1.0000x reference)
import jax
import jax.numpy as jnp
from jax.experimental import pallas as pl


def kernel(user_emb, item_emb, adj_indices, adj_values):
    raise NotImplementedError("write your pallas kernel here")


# speedup vs baseline: 2.7107x; 2.7107x over previous
import functools

import jax
import jax.numpy as jnp
from jax import lax
from jax.experimental import pallas as pl
from jax.experimental.pallas import tpu as pltpu
from jax.experimental.pallas import tpu_sc as plsc

NU = 25000
D = 64
E = 800000
NLAYERS = 3
HP = 25088
NP = 2 * HP
STRIPE = HP // 16
DUMMY = 25080
G = 256
NB = 196
CHUNK = NB * G
EP = 16 * CHUNK
GJ = G // 128


def _layer_body(ego_in, rows_hbm, cols_hbm, vals_hbm, ego_out,
                acc, rowb, colb, lrowb, valb, gbuf, sem):
  c = lax.axis_index("c")
  t = lax.axis_index("s")
  base = c * NU

  def zstore(i, _):
    for u in range(4):
      gbuf[i, pl.ds(u * 16, 16)] = jnp.zeros((16,), jnp.float32)
    return 0
  lax.fori_loop(0, 128, zstore, 0)
  for i in range(STRIPE // 128):
    pltpu.sync_copy(gbuf.at[pl.ds(0, 128)],
                    acc.at[pl.ds(t * STRIPE + i * 128, 128)])
  pltpu.sync_copy(gbuf.at[pl.ds(0, 32)],
                  acc.at[pl.ds(t * STRIPE + (STRIPE // 128) * 128, 32)])
  plsc.subcore_barrier()

  def batch(b, _):
    e0 = pl.multiple_of(t * CHUNK + b * G, G)
    pltpu.sync_copy(cols_hbm.at[pl.ds(pl.multiple_of(e0 // 128, GJ), GJ)], colb)
    descs = [pltpu.async_copy(ego_in.at[colb.at[j]],
                              gbuf.at[pl.ds(j * 128, 128)], sem)
             for j in range(GJ)]
    pltpu.sync_copy(rows_hbm.at[pl.ds(e0, G)], rowb)
    pltpu.sync_copy(vals_hbm.at[pl.ds(e0, G)], valb)
    for k in range(G // 16):
      lr = rowb[pl.ds(k * 16, 16)] - base
      inb = (lr >= 0) & (lr < NU)
      lrowb[k // 8, pl.ds((k % 8) * 16, 16)] = jnp.where(inb, lr, DUMMY)
    for dsc in descs:
      dsc.wait()
    def mul16(k, _):
      vv16 = valb[pl.ds(k * 16, 16)]
      for u in range(16):
        e = k * 16 + u
        vv = lax.gather(
            vv16, jnp.full((16, 1), u, jnp.int32),
            lax.GatherDimensionNumbers(offset_dims=(),
                                       collapsed_slice_dims=(0,),
                                       start_index_map=(0,)),
            slice_sizes=(1,), mode=lax.GatherScatterMode.PROMISE_IN_BOUNDS)
        for d4 in range(4):
          gbuf[e, pl.ds(d4 * 16, 16)] = gbuf[e, pl.ds(d4 * 16, 16)] * vv
      return 0
    lax.fori_loop(0, G // 16, mul16, 0)
    for j in range(GJ):
      pltpu.sync_copy(gbuf.at[pl.ds(j * 128, 128)],
                      acc.at[lrowb.at[j]], add=True)
    return 0
  lax.fori_loop(0, NB, batch, 0)
  plsc.subcore_barrier()

  for i in range(7):
    pltpu.sync_copy(acc.at[pl.ds(t * STRIPE + i * 224, 224)],
                    gbuf.at[pl.ds(0, 224)])
    pltpu.sync_copy(gbuf.at[pl.ds(0, 224)],
                    ego_out.at[pl.ds(c * HP + t * STRIPE + i * 224, 224)])


_layer = functools.partial(
    pl.kernel,
    out_type=jax.ShapeDtypeStruct((NP, D), jnp.float32),
    mesh=plsc.VectorSubcoreMesh(core_axis_name="c", subcore_axis_name="s"),
    compiler_params=pltpu.CompilerParams(use_tc_tiling_on_sc=False),
    scratch_types=[
        pltpu.VMEM_SHARED((HP, D), jnp.float32),
        pltpu.VMEM((G,), jnp.int32),
        pltpu.VMEM((GJ, 128), jnp.int32),
        pltpu.VMEM((GJ, 128), jnp.int32),
        pltpu.VMEM((G,), jnp.float32),
        pltpu.VMEM((G, D), jnp.float32),
        pltpu.SemaphoreType.DMA,
    ],
)(_layer_body)


def _mean_body(a_ref, b_ref, c_ref, o_ref):
  o_ref[...] = (a_ref[...] + b_ref[...] + c_ref[...]) * (1.0 / 3.0)


_mean = pl.pallas_call(
    _mean_body,
    grid=(NP // 1024,),
    in_specs=[pl.BlockSpec((1024, D), lambda i: (i, 0))] * 3,
    out_specs=pl.BlockSpec((1024, D), lambda i: (i, 0)),
    out_shape=jax.ShapeDtypeStruct((NP, D), jnp.float32),
)


def kernel(user_emb, item_emb, adj_indices, adj_values):
  row = adj_indices[0]
  col = adj_indices[1]
  col = col + jnp.where(col >= NU, HP - NU, 0).astype(col.dtype)
  rows_p = jnp.zeros((EP,), jnp.int32).at[:E].set(row.astype(jnp.int32))
  cols_p = (jnp.zeros((EP,), jnp.int32).at[:E].set(col.astype(jnp.int32))
            .reshape(EP // 128, 128))
  vals_p = jnp.zeros((EP,), jnp.float32).at[:E].set(adj_values)
  ego = (jnp.zeros((NP, D), jnp.float32)
         .at[:NU].set(user_emb).at[HP:HP + NU].set(item_emb))
  layers = []
  for _ in range(NLAYERS):
    ego = _layer(ego, rows_p, cols_p, vals_p)
    layers.append(ego)
  mean = _mean(*layers)
  return mean[:NU], mean[HP:HP + NU]


# speedup vs baseline: 3.3738x; 1.2446x over previous
import functools

import jax
import jax.numpy as jnp
from jax import lax
from jax.experimental import pallas as pl
from jax.experimental.pallas import tpu as pltpu
from jax.experimental.pallas import tpu_sc as plsc

NU = 25000
D = 64
E = 800000
NLAYERS = 3
HP = 25088
NP = 2 * HP
STRIPE = HP // 16
DUMMY = 25080
G = 128
SB = 1024
NGB = SB // G
NSB = 49
CHUNK = NSB * SB
EP = 16 * CHUNK


def _layer_body(ego_in, rows_hbm, cols_hbm, vals_hbm, ego_out,
                acc, rowb, colb, lrowb, valb, gbuf, gsem, ssem, isem):
  c = lax.axis_index("c")
  t = lax.axis_index("s")
  base = c * NU

  def zstore(i, _):
    for u in range(4):
      gbuf[0, i, pl.ds(u * 16, 16)] = jnp.zeros((16,), jnp.float32)
    return 0
  lax.fori_loop(0, G, zstore, 0)
  for i in range(STRIPE // G):
    pltpu.sync_copy(gbuf.at[0],
                    acc.at[pl.ds(t * STRIPE + i * G, G)])
  pltpu.sync_copy(gbuf.at[0, pl.ds(0, 32)],
                  acc.at[pl.ds(t * STRIPE + (STRIPE // G) * G, 32)])
  plsc.subcore_barrier()

  def fire_idx(s, p):
    e0 = pl.multiple_of(t * CHUNK + jnp.minimum(s, NSB - 1) * SB, SB)
    pltpu.async_copy(rows_hbm.at[pl.ds(e0, SB)], rowb.at[p], isem)
    pltpu.async_copy(cols_hbm.at[pl.ds(pl.multiple_of(e0 // G, NGB), NGB)],
                     colb.at[p], isem)
    pltpu.async_copy(vals_hbm.at[pl.ds(e0, SB)], valb.at[p], isem)

  def drain_idx(p):
    pltpu.make_async_copy(rows_hbm.at[pl.ds(0, SB)], rowb.at[p], isem).wait()
    pltpu.make_async_copy(cols_hbm.at[pl.ds(0, NGB)], colb.at[p], isem).wait()
    pltpu.make_async_copy(vals_hbm.at[pl.ds(0, SB)], valb.at[p], isem).wait()

  fire_idx(0, 0)

  def super_batch(s, _):
    p = s & 1
    drain_idx(p)
    fire_idx(s + 1, p ^ 1)

    def fire_gather(g):
      return pltpu.async_copy(ego_in.at[colb.at[p, g]],
                              gbuf.at[g & 1], gsem)

    def fire_scatter(g):
      return pltpu.async_copy(gbuf.at[g & 1], acc.at[lrowb.at[g]],
                              ssem, add=True)

    gd = [None] * NGB
    sd = [None] * NGB
    gd[0] = fire_gather(0)
    for g in range(NGB):
      if g < NGB - 1:
        if g >= 1:
          sd[g - 1].wait()
        gd[g + 1] = fire_gather(g + 1)
      for k in range(G // 16):
        lr = rowb[p, pl.ds(g * G + k * 16, 16)] - base
        inb = (lr >= 0) & (lr < NU)
        lrowb[g, pl.ds(k * 16, 16)] = jnp.where(inb, lr, DUMMY)
      gd[g].wait()

      def mul16(k, _):
        vv16 = valb[p, pl.ds(g * G + k * 16, 16)]
        for u in range(16):
          vv = lax.gather(
              vv16, jnp.full((16, 1), u, jnp.int32),
              lax.GatherDimensionNumbers(offset_dims=(),
                                         collapsed_slice_dims=(0,),
                                         start_index_map=(0,)),
              slice_sizes=(1,), mode=lax.GatherScatterMode.PROMISE_IN_BOUNDS)
          e = k * 16 + u
          for d4 in range(4):
            gbuf[g & 1, e, pl.ds(d4 * 16, 16)] = (
                gbuf[g & 1, e, pl.ds(d4 * 16, 16)] * vv)
        return 0
      lax.fori_loop(0, G // 16, mul16, 0)
      sd[g] = fire_scatter(g)
    sd[NGB - 2].wait()
    sd[NGB - 1].wait()
    return 0
  lax.fori_loop(0, NSB, super_batch, 0)
  drain_idx((NSB & 1))
  plsc.subcore_barrier()

  for i in range(STRIPE // G):
    pltpu.sync_copy(acc.at[pl.ds(t * STRIPE + i * G, G)], gbuf.at[0])
    pltpu.sync_copy(gbuf.at[0],
                    ego_out.at[pl.ds(c * HP + t * STRIPE + i * G, G)])
  pltpu.sync_copy(acc.at[pl.ds(t * STRIPE + (STRIPE // G) * G, 32)],
                  gbuf.at[0, pl.ds(0, 32)])
  pltpu.sync_copy(gbuf.at[0, pl.ds(0, 32)],
                  ego_out.at[pl.ds(c * HP + t * STRIPE + (STRIPE // G) * G,
                                   32)])


_layer = functools.partial(
    pl.kernel,
    out_type=jax.ShapeDtypeStruct((NP, D), jnp.float32),
    mesh=plsc.VectorSubcoreMesh(core_axis_name="c", subcore_axis_name="s"),
    compiler_params=pltpu.CompilerParams(use_tc_tiling_on_sc=False),
    scratch_types=[
        pltpu.VMEM_SHARED((HP, D), jnp.float32),
        pltpu.VMEM((2, SB), jnp.int32),
        pltpu.VMEM((2, NGB, G), jnp.int32),
        pltpu.VMEM((NGB, G), jnp.int32),
        pltpu.VMEM((2, SB), jnp.float32),
        pltpu.VMEM((2, G, D), jnp.float32),
        pltpu.SemaphoreType.DMA,
        pltpu.SemaphoreType.DMA,
        pltpu.SemaphoreType.DMA,
    ],
)(_layer_body)


def _mean_body(a_ref, b_ref, c_ref, o_ref):
  o_ref[...] = (a_ref[...] + b_ref[...] + c_ref[...]) * (1.0 / 3.0)


_mean = pl.pallas_call(
    _mean_body,
    grid=(NP // 1024,),
    in_specs=[pl.BlockSpec((1024, D), lambda i: (i, 0))] * 3,
    out_specs=pl.BlockSpec((1024, D), lambda i: (i, 0)),
    out_shape=jax.ShapeDtypeStruct((NP, D), jnp.float32),
)


def kernel(user_emb, item_emb, adj_indices, adj_values):
  row = adj_indices[0]
  col = adj_indices[1]
  col = col + jnp.where(col >= NU, HP - NU, 0).astype(col.dtype)
  rows_p = jnp.zeros((EP,), jnp.int32).at[:E].set(row.astype(jnp.int32))
  cols_p = (jnp.zeros((EP,), jnp.int32).at[:E].set(col.astype(jnp.int32))
            .reshape(EP // G, G))
  vals_p = jnp.zeros((EP,), jnp.float32).at[:E].set(adj_values)
  ego = (jnp.zeros((NP, D), jnp.float32)
         .at[:NU].set(user_emb).at[HP:HP + NU].set(item_emb))
  layers = []
  for _ in range(NLAYERS):
    ego = _layer(ego, rows_p, cols_p, vals_p)
    layers.append(ego)
  mean = _mean(*layers)
  return mean[:NU], mean[HP:HP + NU]


# speedup vs baseline: 4.3280x; 1.2828x over previous
import functools

import jax
import jax.numpy as jnp
from jax import lax
from jax.experimental import pallas as pl
from jax.experimental.pallas import tpu as pltpu
from jax.experimental.pallas import tpu_sc as plsc

NU = 25000
D = 64
W = D // 2
E = 800000
NLAYERS = 3
HP = 25088
NP = 2 * HP
STRIPE = HP // 16
DUMMY = 25080
G = 128
SB = 512
NGB = SB // G
NSB = 98
CHUNK = NSB * SB
EP = 16 * CHUNK


def _bcast(v16, u):
  return lax.gather(
      v16, jnp.full((16, 1), u, jnp.int32),
      lax.GatherDimensionNumbers(offset_dims=(), collapsed_slice_dims=(0,),
                                 start_index_map=(0,)),
      slice_sizes=(1,), mode=lax.GatherScatterMode.PROMISE_IN_BOUNDS)


def _layer_body(ego_in, rows_hbm, cols_hbm, vals_hbm, ego_out, egof_out,
                acc, rowb, colb, lrowb, valb, gb16, sbuf, gsem, ssem, isem):
  c = lax.axis_index("c")
  t = lax.axis_index("s")
  base = c * NU

  def zstore(i, _):
    for u in range(4):
      sbuf[0, i, pl.ds(u * 16, 16)] = jnp.zeros((16,), jnp.float32)
    return 0
  lax.fori_loop(0, G, zstore, 0)
  for i in range(STRIPE // G):
    pltpu.sync_copy(sbuf.at[0], acc.at[pl.ds(t * STRIPE + i * G, G)])
  pltpu.sync_copy(sbuf.at[0, pl.ds(0, 32)],
                  acc.at[pl.ds(t * STRIPE + (STRIPE // G) * G, 32)])
  plsc.subcore_barrier()

  def fire_idx(s, p):
    sc_ = jnp.minimum(s, NSB - 1)
    e0 = pl.multiple_of(t * CHUNK + sc_ * SB, SB)
    pltpu.async_copy(rows_hbm.at[pl.ds(e0, SB)], rowb.at[p], isem)
    pltpu.async_copy(cols_hbm.at[t * NSB + sc_], colb.at[p], isem)
    pltpu.async_copy(vals_hbm.at[pl.ds(e0, SB)], valb.at[p], isem)

  def drain_idx(p):
    pltpu.make_async_copy(rows_hbm.at[pl.ds(0, SB)], rowb.at[p], isem).wait()
    pltpu.make_async_copy(cols_hbm.at[0], colb.at[p], isem).wait()
    pltpu.make_async_copy(vals_hbm.at[pl.ds(0, SB)], valb.at[p], isem).wait()

  fire_idx(0, 0)

  def super_batch(s, _):
    p = s & 1
    drain_idx(p)
    fire_idx(s + 1, p ^ 1)

    def fire_gather(g):
      return pltpu.async_copy(ego_in.at[colb.at[p, g]],
                              gb16.at[g & 1], gsem)

    def fire_scatter(g):
      return pltpu.async_copy(sbuf.at[g & 1], acc.at[lrowb.at[g]],
                              ssem, add=True)

    gd = [None] * NGB
    sd = [None] * NGB
    gd[0] = fire_gather(0)
    for g in range(NGB):
      if g < NGB - 1:
        if g >= 1:
          sd[g - 1].wait()
        gd[g + 1] = fire_gather(g + 1)
      for k in range(G // 16):
        lr = rowb[p, pl.ds(g * G + k * 16, 16)] - base
        inb = (lr >= 0) & (lr < NU)
        lrowb[g, pl.ds(k * 16, 16)] = jnp.where(inb, lr, DUMMY)
      gd[g].wait()

      def mul16(k, _):
        vv16 = valb[p, pl.ds(g * G + k * 16, 16)]
        for u in range(16):
          vv = _bcast(vv16, u)
          e = k * 16 + u
          for h in range(2):
            w_ = gb16[g & 1, e, pl.ds(h * 16, 16)]
            a, b = plsc.unpack(plsc.bitcast(w_, jnp.bfloat16),
                               format=plsc.PackFormat.INTERLEAVED)
            sbuf[g & 1, e, pl.ds(h * 16, 16)] = a * vv
            sbuf[g & 1, e, pl.ds(32 + h * 16, 16)] = b * vv
        return 0
      lax.fori_loop(0, G // 16, mul16, 0)
      sd[g] = fire_scatter(g)
    sd[NGB - 2].wait()
    sd[NGB - 1].wait()
    return 0
  lax.fori_loop(0, NSB, super_batch, 0)
  drain_idx((NSB & 1))
  plsc.subcore_barrier()

  def wb_chunk(r0, nrows):
    pltpu.sync_copy(acc.at[pl.ds(t * STRIPE + r0, nrows)],
                    sbuf.at[0, pl.ds(0, nrows)])
    pltpu.sync_copy(sbuf.at[0, pl.ds(0, nrows)],
                    egof_out.at[pl.ds(c * HP + t * STRIPE + r0, nrows)])
    def prow(r, _):
      for h in range(2):
        a = sbuf[0, r, pl.ds(h * 16, 16)]
        b = sbuf[0, r, pl.ds(32 + h * 16, 16)]
        packed = plsc.pack(a, b, format=plsc.PackFormat.INTERLEAVED)
        gb16[0, r, pl.ds(h * 16, 16)] = plsc.bitcast(packed, jnp.int32)
      return 0
    lax.fori_loop(0, nrows, prow, 0)
    pltpu.sync_copy(gb16.at[0, pl.ds(0, nrows)],
                    ego_out.at[pl.ds(c * HP + t * STRIPE + r0, nrows)])
  for i in range(STRIPE // G):
    wb_chunk(i * G, G)
  wb_chunk((STRIPE // G) * G, 32)


_layer = functools.partial(
    pl.kernel,
    out_type=(jax.ShapeDtypeStruct((NP, W), jnp.int32),
              jax.ShapeDtypeStruct((NP, D), jnp.float32)),
    mesh=plsc.VectorSubcoreMesh(core_axis_name="c", subcore_axis_name="s"),
    compiler_params=pltpu.CompilerParams(use_tc_tiling_on_sc=False,
                                         needs_layout_passes=False),
    scratch_types=[
        pltpu.VMEM_SHARED((HP, D), jnp.float32),
        pltpu.VMEM((2, SB), jnp.int32),
        pltpu.VMEM((2, NGB, G), jnp.int32),
        pltpu.VMEM((NGB, G), jnp.int32),
        pltpu.VMEM((2, SB), jnp.float32),
        pltpu.VMEM((2, G, W), jnp.int32),
        pltpu.VMEM((2, G, D), jnp.float32),
        pltpu.SemaphoreType.DMA,
        pltpu.SemaphoreType.DMA,
        pltpu.SemaphoreType.DMA,
    ],
)(_layer_body)


def _mean_body(a_ref, b_ref, c_ref, o_ref):
  o_ref[...] = (a_ref[...] + b_ref[...] + c_ref[...]) * (1.0 / 3.0)


_mean = pl.pallas_call(
    _mean_body,
    grid=(NP // 1024,),
    in_specs=[pl.BlockSpec((1024, D), lambda i: (i, 0))] * 3,
    out_specs=pl.BlockSpec((1024, D), lambda i: (i, 0)),
    out_shape=jax.ShapeDtypeStruct((NP, D), jnp.float32),
)


def _pack_rows(x):
  z = jnp.stack([x[:, :W], x[:, W:]], axis=-1).reshape(x.shape[0], D)
  b = z.astype(jnp.bfloat16).reshape(x.shape[0], W, 2)
  return lax.bitcast_convert_type(b, jnp.int32)


def kernel(user_emb, item_emb, adj_indices, adj_values):
  row = adj_indices[0]
  col = adj_indices[1]
  col = col + jnp.where(col >= NU, HP - NU, 0).astype(col.dtype)
  rows_p = jnp.zeros((EP,), jnp.int32).at[:E].set(row.astype(jnp.int32))
  cols_p = (jnp.zeros((EP,), jnp.int32).at[:E].set(col.astype(jnp.int32))
            .reshape(EP // SB, NGB, G))
  vals_p = jnp.zeros((EP,), jnp.float32).at[:E].set(adj_values)
  ego0 = (jnp.zeros((NP, D), jnp.float32)
          .at[:NU].set(user_emb).at[HP:HP + NU].set(item_emb))
  ego16 = _pack_rows(ego0)
  layers = []
  for _ in range(NLAYERS):
    ego16, egof = _layer(ego16, rows_p, cols_p, vals_p)
    layers.append(egof)
  mean = _mean(*layers)
  return mean[:NU], mean[HP:HP + NU]


# speedup vs baseline: 7.8604x; 1.8162x over previous
import functools

import jax
import jax.numpy as jnp
from jax import lax
from jax.experimental import pallas as pl
from jax.experimental.pallas import tpu as pltpu
from jax.experimental.pallas import tpu_sc as plsc

NU = 25000
D = 64
E = 800000
NLAYERS = 3
HP = 25088
NP = 2 * HP
STRIPE = HP // 16
DUMMY = 25080
G = 256
SB = 1024
NGB = SB // G
NSB = 49
CHUNK = NSB * SB
EP = 16 * CHUNK
WBC = 256


def _bcast(v16, u):
  return lax.gather(
      v16, jnp.full((16, 1), u, jnp.int32),
      lax.GatherDimensionNumbers(offset_dims=(), collapsed_slice_dims=(0,),
                                 start_index_map=(0,)),
      slice_sizes=(1,), mode=lax.GatherScatterMode.PROMISE_IN_BOUNDS)


def _layer_body(ego_in, rows_hbm, cols_hbm, vals_hbm, ego_out, egof_out,
                acc, rowb, colb, lrowb, valb, gb16, wbf, gsem, ssem, isem):
  c = lax.axis_index("c")
  t = lax.axis_index("s")
  base = c * NU

  def zstore(i, _):
    for u in range(2):
      gb16[0, i, pl.ds(u * 32, 32)] = jnp.zeros((32,), jnp.bfloat16)
    return 0
  lax.fori_loop(0, WBC, zstore, 0)
  for i in range(STRIPE // WBC):
    pltpu.sync_copy(gb16.at[0], acc.at[pl.ds(t * STRIPE + i * WBC, WBC)])
  pltpu.sync_copy(gb16.at[0, pl.ds(0, 32)],
                  acc.at[pl.ds(t * STRIPE + (STRIPE // WBC) * WBC, 32)])
  plsc.subcore_barrier()

  def fire_idx(s, p):
    sc_ = jnp.minimum(s, NSB - 1)
    e0 = pl.multiple_of(t * CHUNK + sc_ * SB, SB)
    pltpu.async_copy(rows_hbm.at[pl.ds(e0, SB)], rowb.at[p], isem)
    pltpu.async_copy(cols_hbm.at[t * NSB + sc_], colb.at[p], isem)
    pltpu.async_copy(vals_hbm.at[pl.ds(e0, SB)], valb.at[p], isem)

  def drain_idx(p):
    pltpu.make_async_copy(rows_hbm.at[pl.ds(0, SB)], rowb.at[p], isem).wait()
    pltpu.make_async_copy(cols_hbm.at[0], colb.at[p], isem).wait()
    pltpu.make_async_copy(vals_hbm.at[pl.ds(0, SB)], valb.at[p], isem).wait()

  fire_idx(0, 0)

  def super_batch(s, _):
    p = s & 1
    drain_idx(p)
    fire_idx(s + 1, p ^ 1)

    def fire_gather(g):
      return [pltpu.async_copy(ego_in.at[colb.at[p, 2 * g + h]],
                               gb16.at[g, pl.ds(h * 128, 128)], gsem)
              for h in range(2)]

    def fire_scatter(g):
      return [pltpu.async_copy(gb16.at[g, pl.ds(h * 128, 128)],
                               acc.at[lrowb.at[2 * g + h]], ssem, add=True)
              for h in range(2)]

    gd = [fire_gather(g) for g in range(NGB)]
    sd = [None] * NGB
    for g in range(NGB):
      for k in range(G // 16):
        lr = rowb[p, pl.ds(g * G + k * 16, 16)] - base
        inb = (lr >= 0) & (lr < NU)
        lrowb[2 * g + k // 8, pl.ds((k % 8) * 16, 16)] = (
            jnp.where(inb, lr, DUMMY))
      for d_ in gd[g]:
        d_.wait()

      def mul16(k, _):
        vv16 = valb[p, pl.ds(g * G + k * 16, 16)]
        for u in range(16):
          vvb = plsc.pack(_bcast(vv16, u), _bcast(vv16, u),
                          format=plsc.PackFormat.INTERLEAVED)
          e = k * 16 + u
          for h in range(2):
            gb16[g, e, pl.ds(h * 32, 32)] = (
                gb16[g, e, pl.ds(h * 32, 32)] * vvb)
        return 0
      lax.fori_loop(0, G // 16, mul16, 0)
      sd[g] = fire_scatter(g)
    for g in range(NGB):
      for d_ in sd[g]:
        d_.wait()
    return 0
  lax.fori_loop(0, NSB, super_batch, 0)
  drain_idx((NSB & 1))
  plsc.subcore_barrier()

  def wb_chunk(r0, nrows):
    pltpu.sync_copy(acc.at[pl.ds(t * STRIPE + r0, nrows)],
                    gb16.at[0, pl.ds(0, nrows)])
    pltpu.sync_copy(gb16.at[0, pl.ds(0, nrows)],
                    ego_out.at[pl.ds(c * HP + t * STRIPE + r0, nrows)])
    def prow(r, _):
      for h in range(2):
        a, b = plsc.unpack(gb16[0, r, pl.ds(h * 32, 32)],
                           format=plsc.PackFormat.INTERLEAVED)
        wbf[r, pl.ds(h * 16, 16)] = a
        wbf[r, pl.ds(32 + h * 16, 16)] = b
      return 0
    lax.fori_loop(0, nrows, prow, 0)
    pltpu.sync_copy(wbf.at[pl.ds(0, nrows)],
                    egof_out.at[pl.ds(c * HP + t * STRIPE + r0, nrows)])
  for i in range(STRIPE // WBC):
    wb_chunk(i * WBC, WBC)
  wb_chunk((STRIPE // WBC) * WBC, 32)


_layer = functools.partial(
    pl.kernel,
    out_type=(jax.ShapeDtypeStruct((NP, D), jnp.bfloat16),
              jax.ShapeDtypeStruct((NP, D), jnp.float32)),
    mesh=plsc.VectorSubcoreMesh(core_axis_name="c", subcore_axis_name="s"),
    compiler_params=pltpu.CompilerParams(use_tc_tiling_on_sc=False,
                                         needs_layout_passes=False),
    scratch_types=[
        pltpu.VMEM_SHARED((HP, D), jnp.bfloat16),
        pltpu.VMEM((2, SB), jnp.int32),
        pltpu.VMEM((2, 2 * NGB, 128), jnp.int32),
        pltpu.VMEM((2 * NGB, 128), jnp.int32),
        pltpu.VMEM((2, SB), jnp.float32),
        pltpu.VMEM((NGB, G, D), jnp.bfloat16),
        pltpu.VMEM((WBC, D), jnp.float32),
        pltpu.SemaphoreType.DMA,
        pltpu.SemaphoreType.DMA,
        pltpu.SemaphoreType.DMA,
    ],
)(_layer_body)


def _mean_body(a_ref, b_ref, c_ref, o_ref):
  o_ref[...] = (a_ref[...] + b_ref[...] + c_ref[...]) * (1.0 / 3.0)


_mean = pl.pallas_call(
    _mean_body,
    grid=(NP // 1024,),
    in_specs=[pl.BlockSpec((1024, D), lambda i: (i, 0))] * 3,
    out_specs=pl.BlockSpec((1024, D), lambda i: (i, 0)),
    out_shape=jax.ShapeDtypeStruct((NP, D), jnp.float32),
)


def kernel(user_emb, item_emb, adj_indices, adj_values):
  row = adj_indices[0]
  col = adj_indices[1]
  col = col + jnp.where(col >= NU, HP - NU, 0).astype(col.dtype)
  rows_p = jnp.zeros((EP,), jnp.int32).at[:E].set(row.astype(jnp.int32))
  cols_p = (jnp.zeros((EP,), jnp.int32).at[:E].set(col.astype(jnp.int32))
            .reshape(EP // SB, 2 * NGB, 128))
  vals_p = jnp.zeros((EP,), jnp.float32).at[:E].set(adj_values)
  ego0 = (jnp.zeros((NP, D), jnp.float32)
          .at[:NU].set(user_emb).at[HP:HP + NU].set(item_emb))
  half = D // 2
  ego16 = (jnp.stack([ego0[:, :half], ego0[:, half:]], axis=-1)
           .reshape(NP, D).astype(jnp.bfloat16))
  layers = []
  for _ in range(NLAYERS):
    ego16, egof = _layer(ego16, rows_p, cols_p, vals_p)
    layers.append(egof)
  mean = _mean(*layers)
  return mean[:NU], mean[HP:HP + NU]


# speedup vs baseline: 9.1718x; 1.1668x over previous
import functools

import jax
import jax.numpy as jnp
from jax import lax
from jax.experimental import pallas as pl
from jax.experimental.pallas import tpu as pltpu
from jax.experimental.pallas import tpu_sc as plsc

NU = 25000
D = 64
E = 800000
NLAYERS = 3
HP = 25088
NP = 2 * HP
STRIPE = HP // 16
DUMMY = 25080
G = 256
SB = 1024
NGB = SB // G
NSB = 49
NSLOT = NSB + 1
CHUNK = NSB * SB
EP = 16 * CHUNK
WBC = 256
OCAP = 2 * SB + 16


def _bcast(v16, u):
  return lax.gather(
      v16, jnp.full((16, 1), u, jnp.int32),
      lax.GatherDimensionNumbers(offset_dims=(), collapsed_slice_dims=(0,),
                                 start_index_map=(0,)),
      slice_sizes=(1,), mode=lax.GatherScatterMode.PROMISE_IN_BOUNDS)


def _sum16(v):
  return lax.reduce_sum_p.bind(v, axes=(0,))


def _compact_body(rows_hbm, cols_hbm, vals_hbm, lrowc, colc, valc, counts,
                  rowb, colb, valb, olr, oco, ova, cntv, isem, osem):
  c = lax.axis_index("c")
  t = lax.axis_index("s")
  base = c * NU

  def fire_idx(s, p):
    sc_ = jnp.minimum(s, NSB - 1)
    e0 = pl.multiple_of(t * CHUNK + sc_ * SB, SB)
    pltpu.async_copy(rows_hbm.at[pl.ds(e0, SB)], rowb.at[p], isem)
    pltpu.async_copy(cols_hbm.at[pl.ds(e0, SB)], colb.at[p], isem)
    pltpu.async_copy(vals_hbm.at[pl.ds(e0, SB)], valb.at[p], isem)

  def drain_idx(p):
    pltpu.make_async_copy(rows_hbm.at[pl.ds(0, SB)], rowb.at[p], isem).wait()
    pltpu.make_async_copy(cols_hbm.at[pl.ds(0, SB)], colb.at[p], isem).wait()
    pltpu.make_async_copy(vals_hbm.at[pl.ds(0, SB)], valb.at[p], isem).wait()

  def flush(ns):
    ds_ = []
    for k in range(SB // 128):
      ds_.append(pltpu.async_copy(olr.at[pl.ds(k * 128, 128)],
                                  lrowc.at[c, t, ns, k], osem))
      ds_.append(pltpu.async_copy(oco.at[pl.ds(k * 128, 128)],
                                  colc.at[c, t, ns, k], osem))
      ds_.append(pltpu.async_copy(ova.at[pl.ds(k * 128, 128)],
                                  valc.at[c, t, ns, pl.ds(k * 128, 128)],
                                  osem))
    for d_ in ds_:
      d_.wait()

  fire_idx(0, 0)

  def super_batch(s, carry):
    cnt, ns = carry
    p = s & 1
    drain_idx(p)
    fire_idx(s + 1, p ^ 1)

    def grp(k, cnt):
      lr = rowb[p, pl.ds(k * 16, 16)] - base
      inb = (lr >= 0) & (lr < NU)
      plsc.store_compressed(olr.at[pl.ds(cnt, 16)], lr, mask=inb)
      plsc.store_compressed(oco.at[pl.ds(cnt, 16)],
                            colb[p, pl.ds(k * 16, 16)], mask=inb)
      plsc.store_compressed(ova.at[pl.ds(cnt, 16)],
                            valb[p, pl.ds(k * 16, 16)], mask=inb)
      return cnt + _sum16(jnp.where(inb, 1, 0))
    cnt = lax.fori_loop(0, SB // 16, grp, cnt)

    full = cnt >= SB
    @pl.when(full)
    def _():
      flush(ns)
      rem = cnt - SB
      def mv(i, _):
        for b_ in (olr, oco, ova):
          b_[pl.ds(i * 16, 16)] = b_[pl.ds(SB + i * 16, 16)]
        return 0
      lax.fori_loop(0, (rem + 15) >> 4, mv, 0)
    cnt = jnp.where(full, cnt - SB, cnt)
    ns = ns + jnp.where(full, 1, 0)
    return (cnt, ns)

  cnt, ns = lax.fori_loop(0, NSB, super_batch,
                          (jnp.int32(0), jnp.int32(0)))
  drain_idx(NSB & 1)

  olr[pl.ds(cnt, 16)] = jnp.full((16,), DUMMY, jnp.int32)
  oco[pl.ds(cnt, 16)] = jnp.zeros((16,), jnp.int32)
  ova[pl.ds(cnt, 16)] = jnp.zeros((16,), jnp.float32)
  cnt_al = (cnt + 15) & ~15
  rem_to = (SB - (cnt_al & (SB - 1))) & (SB - 1)
  def pad(i, _):
    olr[pl.ds(cnt_al + i * 16, 16)] = jnp.full((16,), DUMMY, jnp.int32)
    oco[pl.ds(cnt_al + i * 16, 16)] = jnp.zeros((16,), jnp.int32)
    ova[pl.ds(cnt_al + i * 16, 16)] = jnp.zeros((16,), jnp.float32)
    return 0
  lax.fori_loop(0, rem_to >> 4, pad, 0)
  cnt_f = cnt_al + rem_to
  @pl.when(cnt_f > 0)
  def _():
    flush(ns)
  ns = ns + jnp.where(cnt_f > 0, 1, 0)

  cntv[pl.ds(0, 16)] = jnp.broadcast_to(ns, (16,))
  pltpu.sync_copy(cntv, counts.at[c, t])


_compact = functools.partial(
    pl.kernel,
    out_type=(jax.ShapeDtypeStruct((2, 16, NSLOT, SB // 128, 128), jnp.int32),
              jax.ShapeDtypeStruct((2, 16, NSLOT, SB // 128, 128), jnp.int32),
              jax.ShapeDtypeStruct((2, 16, NSLOT, SB), jnp.float32),
              jax.ShapeDtypeStruct((2, 16, 16), jnp.int32)),
    mesh=plsc.VectorSubcoreMesh(core_axis_name="c", subcore_axis_name="s"),
    compiler_params=pltpu.CompilerParams(use_tc_tiling_on_sc=False,
                                         needs_layout_passes=False),
    scratch_types=[
        pltpu.VMEM((2, SB), jnp.int32),
        pltpu.VMEM((2, SB), jnp.int32),
        pltpu.VMEM((2, SB), jnp.float32),
        pltpu.VMEM((OCAP,), jnp.int32),
        pltpu.VMEM((OCAP,), jnp.int32),
        pltpu.VMEM((OCAP,), jnp.float32),
        pltpu.VMEM((16,), jnp.int32),
        pltpu.SemaphoreType.DMA,
        pltpu.SemaphoreType.DMA,
    ],
)(_compact_body)


def _layer_body(ego_in, lrowc, colc, valc, counts, ego_out, egof_out,
                acc, cntb, colb, lrowb, valb, gb16, wbf, gsem, ssem, isem):
  c = lax.axis_index("c")
  t = lax.axis_index("s")

  pltpu.sync_copy(counts.at[c, t], cntb)
  nt = lax.reduce_max_p.bind(cntb[pl.ds(0, 16)], axes=(0,))

  def zstore(i, _):
    for u in range(2):
      gb16[0, i, pl.ds(u * 32, 32)] = jnp.zeros((32,), jnp.bfloat16)
    return 0
  lax.fori_loop(0, WBC, zstore, 0)
  for i in range(STRIPE // WBC):
    pltpu.sync_copy(gb16.at[0], acc.at[pl.ds(t * STRIPE + i * WBC, WBC)])
  pltpu.sync_copy(gb16.at[0, pl.ds(0, 32)],
                  acc.at[pl.ds(t * STRIPE + (STRIPE // WBC) * WBC, 32)])
  plsc.subcore_barrier()

  def fire_idx(s, p):
    sc_ = jnp.clip(s, 0, jnp.maximum(nt - 1, 0))
    pltpu.async_copy(lrowc.at[c, t, sc_], lrowb.at[p], isem)
    pltpu.async_copy(colc.at[c, t, sc_], colb.at[p], isem)
    pltpu.async_copy(valc.at[c, t, sc_], valb.at[p], isem)

  def drain_idx(p):
    pltpu.make_async_copy(lrowc.at[0, 0, 0], lrowb.at[p], isem).wait()
    pltpu.make_async_copy(colc.at[0, 0, 0], colb.at[p], isem).wait()
    pltpu.make_async_copy(valc.at[0, 0, 0], valb.at[p], isem).wait()

  fire_idx(0, 0)

  def super_batch(s, _):
    p = s & 1
    drain_idx(p)
    fire_idx(s + 1, p ^ 1)

    def fire_gather(g):
      return [pltpu.async_copy(ego_in.at[colb.at[p, 2 * g + h]],
                               gb16.at[g, pl.ds(h * 128, 128)], gsem)
              for h in range(2)]

    def fire_scatter(g):
      return [pltpu.async_copy(gb16.at[g, pl.ds(h * 128, 128)],
                               acc.at[lrowb.at[p, 2 * g + h]], ssem,
                               add=True)
              for h in range(2)]

    gd = [fire_gather(g) for g in range(NGB)]
    sd = [None] * NGB
    for g in range(NGB):
      for d_ in gd[g]:
        d_.wait()
      def mul16(k, _):
        vv16 = valb[p, pl.ds(g * G + k * 16, 16)]
        for u in range(16):
          vvb = plsc.pack(_bcast(vv16, u), _bcast(vv16, u),
                          format=plsc.PackFormat.INTERLEAVED)
          e = k * 16 + u
          for h in range(2):
            gb16[g, e, pl.ds(h * 32, 32)] = (
                gb16[g, e, pl.ds(h * 32, 32)] * vvb)
        return 0
      lax.fori_loop(0, G // 16, mul16, 0)
      sd[g] = fire_scatter(g)
    for g in range(NGB):
      for d_ in sd[g]:
        d_.wait()
    return 0
  lax.fori_loop(0, nt, super_batch, 0)
  drain_idx(nt & 1)
  plsc.subcore_barrier()

  def wb_chunk(r0, nrows):
    pltpu.sync_copy(acc.at[pl.ds(t * STRIPE + r0, nrows)],
                    gb16.at[0, pl.ds(0, nrows)])
    pltpu.sync_copy(gb16.at[0, pl.ds(0, nrows)],
                    ego_out.at[pl.ds(c * HP + t * STRIPE + r0, nrows)])
    def prow(r, _):
      for h in range(2):
        a, b = plsc.unpack(gb16[0, r, pl.ds(h * 32, 32)],
                           format=plsc.PackFormat.INTERLEAVED)
        wbf[r, pl.ds(h * 16, 16)] = a
        wbf[r, pl.ds(32 + h * 16, 16)] = b
      return 0
    lax.fori_loop(0, nrows, prow, 0)
    pltpu.sync_copy(wbf.at[pl.ds(0, nrows)],
                    egof_out.at[pl.ds(c * HP + t * STRIPE + r0, nrows)])
  for i in range(STRIPE // WBC):
    wb_chunk(i * WBC, WBC)
  wb_chunk((STRIPE // WBC) * WBC, 32)


_layer = functools.partial(
    pl.kernel,
    out_type=(jax.ShapeDtypeStruct((NP, D), jnp.bfloat16),
              jax.ShapeDtypeStruct((NP, D), jnp.float32)),
    mesh=plsc.VectorSubcoreMesh(core_axis_name="c", subcore_axis_name="s"),
    compiler_params=pltpu.CompilerParams(use_tc_tiling_on_sc=False,
                                         needs_layout_passes=False),
    scratch_types=[
        pltpu.VMEM_SHARED((HP, D), jnp.bfloat16),
        pltpu.VMEM((16,), jnp.int32),
        pltpu.VMEM((2, 2 * NGB, 128), jnp.int32),
        pltpu.VMEM((2, 2 * NGB, 128), jnp.int32),
        pltpu.VMEM((2, SB), jnp.float32),
        pltpu.VMEM((NGB, G, D), jnp.bfloat16),
        pltpu.VMEM((WBC, D), jnp.float32),
        pltpu.SemaphoreType.DMA,
        pltpu.SemaphoreType.DMA,
        pltpu.SemaphoreType.DMA,
    ],
)(_layer_body)


def _mean_body(a_ref, b_ref, c_ref, o_ref):
  o_ref[...] = (a_ref[...] + b_ref[...] + c_ref[...]) * (1.0 / 3.0)


_mean = pl.pallas_call(
    _mean_body,
    grid=(NP // 1024,),
    in_specs=[pl.BlockSpec((1024, D), lambda i: (i, 0))] * 3,
    out_specs=pl.BlockSpec((1024, D), lambda i: (i, 0)),
    out_shape=jax.ShapeDtypeStruct((NP, D), jnp.float32),
)


def kernel(user_emb, item_emb, adj_indices, adj_values):
  row = adj_indices[0]
  col = adj_indices[1]
  col = col + jnp.where(col >= NU, HP - NU, 0).astype(col.dtype)
  rows_p = jnp.zeros((EP,), jnp.int32).at[:E].set(row.astype(jnp.int32))
  cols_p = jnp.zeros((EP,), jnp.int32).at[:E].set(col.astype(jnp.int32))
  vals_p = jnp.zeros((EP,), jnp.float32).at[:E].set(adj_values)
  ego0 = (jnp.zeros((NP, D), jnp.float32)
          .at[:NU].set(user_emb).at[HP:HP + NU].set(item_emb))
  half = D // 2
  ego16 = (jnp.stack([ego0[:, :half], ego0[:, half:]], axis=-1)
           .reshape(NP, D).astype(jnp.bfloat16))
  lrowc, colc, valc, counts = _compact(rows_p, cols_p, vals_p)
  layers = []
  for _ in range(NLAYERS):
    ego16, egof = _layer(ego16, lrowc, colc, valc, counts)
    layers.append(egof)
  mean = _mean(*layers)
  return mean[:NU], mean[HP:HP + NU]
